# baseline (speedup 1.0000x reference)
import functools

import numpy as np
import jax
import jax.numpy as jnp
from jax import lax
from jax.experimental import pallas as pl
from jax.experimental.pallas import tpu as pltpu
from jax.experimental.pallas import tpu_sc as plsc

_INPUT_SIZE = 384
_N_FRAMES = 256
_N_LM = 227
_COL_B = 6
_SEG = (0, 160, 181, 206, 227)
_KPF = 2 * _N_LM
_KPAD = 464
_L = 16
_NCHUNK = _KPAD // _L
_RSLAB = 232
_CSLAB = 128
_CLIVE = 16
_GCHUNK = 128

_NC = 2
_NS = 16
_NW = _NC * _NS
_FPT = _N_FRAMES // _NW
_KPT = _FPT * _KPAD
_NGC = _KPT // _GCHUNK

_CTOT = _RSLAB * 2 * _N_FRAMES

_RUNS = tuple((2 * _SEG[s], 2 * _SEG[s + 1]) for s in range(4))


def _chunk_pieces(ch):
    pieces = []
    for s, (lo, hi) in enumerate(_RUNS):
        a = max(lo, _L * ch) - _L * ch
        b = min(hi, _L * ch + _L) - _L * ch
        if a < b:
            pieces.append((s, a, b))
    return pieces


def _build_offsets():
    f = np.arange(_N_FRAMES, dtype=np.int64)[:, None]
    k = np.arange(_KPAD, dtype=np.int64)[None, :]
    r = np.minimum(k, _KPF - 1) // 2
    cp = np.where(k < _KPF, k % 2, 0)
    o = r * (2 * _N_FRAMES) + cp * _N_FRAMES + f
    o = np.where(k < _KPF, o, 0)
    return o.astype(np.int32).reshape(_NW, _NGC, _GCHUNK)


def _rsqrt_newton(s):
    i = lax.bitcast_convert_type(s, jnp.int32)
    y = lax.bitcast_convert_type(np.int32(0x5F3759DF) - (i >> 1), jnp.float32)
    for _ in range(3):
        y = y * (1.5 - 0.5 * s * y * y)
    return jnp.where(s == 0.0, 1.0, y)


def _compact_tc_body(x_ref, o_ref):
    x = x_ref[...]
    y = jnp.stack([x[:, 0, :], x[:, _COL_B, :]], axis=1)
    t = y.reshape(29, 8, 2, 16, 16)
    t = jnp.transpose(t, (2, 3, 0, 1, 4))
    o_ref[...] = t.reshape(32, 29, 128)


@functools.cache
def _make_compact_tc():
    return pl.pallas_call(
        _compact_tc_body,
        grid=(1,),
        in_specs=[pl.BlockSpec((_RSLAB, 8, _N_FRAMES), lambda i: (0, 0, 0))],
        out_specs=pl.BlockSpec((32, 29, 128), lambda i: (0, 0, 0)),
        out_shape=jax.ShapeDtypeStruct((32, 29, 128), jnp.float32),
    )


_NRB = 32


def _seg_of(r):
    for s in range(4):
        if _SEG[s] <= r < _SEG[s + 1]:
            return s
    return None


@functools.cache
def _make_preprocess_sc():
    return pl.kernel(
        _preprocess_sc_body,
        out_type=jax.ShapeDtypeStruct((_INPUT_SIZE, _KPAD), jnp.float32),
        mesh=plsc.VectorSubcoreMesh(core_axis_name="c", subcore_axis_name="s",
                                    num_cores=_NC, num_subcores=_NS),
        scratch_types=[
            pltpu.VMEM((2 * _NRB, 128), jnp.float32),
            pltpu.VMEM((_FPT, _KPAD), jnp.float32),
            pltpu.VMEM_SHARED((_NS, 29, 128), jnp.float32),
        ],
        compiler_params=pltpu.CompilerParams(needs_layout_passes=False),
    )


def _preprocess_sc_body(compact_hbm, out_hbm, myv_v, vals_v, spm):
    core = lax.axis_index("c")
    sid = lax.axis_index("s")
    cp = sid & 1
    g = core * 8 + (sid >> 1)
    w_pack = cp * 16 + g

    own = myv_v.at[pl.ds(0, 29)]
    pltpu.sync_copy(compact_hbm.at[w_pack], own)

    lane_iota = lax.iota(jnp.int32, 16)
    even = (lane_iota & 1) == 0
    zeros = jnp.zeros((16,), jnp.float32)

    accs = [zeros, zeros, zeros, zeros]
    for r in range(_N_LM):
        v = myv_v[r >> 3, pl.ds((r & 7) * _L, _L)]
        accs[_seg_of(r)] = accs[_seg_of(r)] + v * v
    rvecs = [_rsqrt_newton(a) for a in accs]

    for r in range(_N_LM):
        sl = (r >> 3, pl.ds((r & 7) * _L, _L))
        y = myv_v[sl] * rvecs[_seg_of(r)]
        myv_v[sl] = jnp.where(y != y, 0.0, y)

    @pl.when(cp == 1)
    def _():
        pltpu.sync_copy(own, spm.at[sid])

    plsc.subcore_barrier()

    @pl.when(cp == 0)
    def _():
        pltpu.sync_copy(spm.at[sid + 1], myv_v.at[pl.ds(_NRB, 29)])

        def frame_body(fl, _):
            for ch in range(_NCHUNK):
                kvec = _L * ch + lane_iota
                r = kvec >> 1
                rows = (r >> 3) + (kvec & 1) * _NRB
                lanes = (r & 7) * _L + fl
                vals_v[fl & 7, pl.ds(_L * ch, _L)] = plsc.load_gather(
                    myv_v, [rows, lanes])
            return ()

        lax.fori_loop(0, _FPT, frame_body, (), unroll=False)
        pltpu.sync_copy(vals_v, out_hbm.at[pl.ds(16 * g, _FPT)])
        lax.fori_loop(_FPT, 2 * _FPT, frame_body, (), unroll=False)
        pltpu.sync_copy(vals_v, out_hbm.at[pl.ds(16 * g + _FPT, _FPT)])

    @pl.when(cp == 1)
    def _():
        neg1 = jnp.full((16,), -1.0, jnp.float32)
        for r in range(_FPT):
            for ch in range(_NCHUNK):
                vals_v[r, pl.ds(_L * ch, _L)] = neg1
        pltpu.sync_copy(vals_v,
                        out_hbm.at[pl.ds(_N_FRAMES + g * _FPT, _FPT)])


def kernel(data):
    n_frames = data.shape[0]
    data_t = jnp.transpose(data, (1, 2, 0))
    compact = _make_compact_tc()(data_t)
    padded = _make_preprocess_sc()(compact)
    out = padded[:, :_KPF]
    idxs = jnp.concatenate(
        [jnp.arange(n_frames, dtype=jnp.int32),
         jnp.full((_INPUT_SIZE - n_frames,), -1, dtype=jnp.int32)]
    )
    return (out, idxs)


# speedup vs baseline: 1.0686x; 1.0686x over previous
import functools

import numpy as np
import jax
import jax.numpy as jnp
from jax import lax
from jax.experimental import pallas as pl
from jax.experimental.pallas import tpu as pltpu
from jax.experimental.pallas import tpu_sc as plsc

_INPUT_SIZE = 384
_N_FRAMES = 256
_N_LM = 227
_COL_B = 6
_SEG = (0, 160, 181, 206, 227)
_KPF = 2 * _N_LM
_KPAD = 464
_L = 16
_NCHUNK = _KPAD // _L
_RSLAB = 232
_CSLAB = 128
_CLIVE = 16
_GCHUNK = 128

_NC = 2
_NS = 16
_NW = _NC * _NS
_FPT = _N_FRAMES // _NW
_KPT = _FPT * _KPAD
_NGC = _KPT // _GCHUNK

_CTOT = _RSLAB * 2 * _N_FRAMES

_RUNS = tuple((2 * _SEG[s], 2 * _SEG[s + 1]) for s in range(4))


def _chunk_pieces(ch):
    pieces = []
    for s, (lo, hi) in enumerate(_RUNS):
        a = max(lo, _L * ch) - _L * ch
        b = min(hi, _L * ch + _L) - _L * ch
        if a < b:
            pieces.append((s, a, b))
    return pieces


def _build_offsets():
    f = np.arange(_N_FRAMES, dtype=np.int64)[:, None]
    k = np.arange(_KPAD, dtype=np.int64)[None, :]
    r = np.minimum(k, _KPF - 1) // 2
    cp = np.where(k < _KPF, k % 2, 0)
    o = r * (2 * _N_FRAMES) + cp * _N_FRAMES + f
    o = np.where(k < _KPF, o, 0)
    return o.astype(np.int32).reshape(_NW, _NGC, _GCHUNK)


def _rsqrt_newton(s):
    i = lax.bitcast_convert_type(s, jnp.int32)
    y = lax.bitcast_convert_type(np.int32(0x5F3759DF) - (i >> 1), jnp.float32)
    for _ in range(3):
        y = y * (1.5 - 0.5 * s * y * y)
    return jnp.where(s == 0.0, 1.0, y)


def _compact_tc_body(x_ref, o_ref):
    x = x_ref[...]
    y = jnp.stack([x[:, 0, :], x[:, _COL_B, :]], axis=1)
    t = y.reshape(29, 8, 2, 16, 16)
    t = jnp.transpose(t, (2, 3, 0, 1, 4))
    o_ref[...] = t.reshape(32, 29, 128)


@functools.cache
def _make_compact_tc():
    return pl.pallas_call(
        _compact_tc_body,
        grid=(1,),
        in_specs=[pl.BlockSpec((_RSLAB, 8, _N_FRAMES), lambda i: (0, 0, 0))],
        out_specs=pl.BlockSpec((32, 29, 128), lambda i: (0, 0, 0)),
        out_shape=jax.ShapeDtypeStruct((32, 29, 128), jnp.float32),
    )


_NRB = 32


def _seg_of(r):
    for s in range(4):
        if _SEG[s] <= r < _SEG[s + 1]:
            return s
    return None


@functools.cache
def _make_preprocess_sc():
    return pl.kernel(
        _preprocess_sc_body,
        out_type=jax.ShapeDtypeStruct((_INPUT_SIZE, _KPAD), jnp.float32),
        mesh=plsc.VectorSubcoreMesh(core_axis_name="c", subcore_axis_name="s",
                                    num_cores=_NC, num_subcores=_NS),
        scratch_types=[
            pltpu.VMEM((2 * _NRB, 128), jnp.float32),
            pltpu.VMEM((_FPT, _KPAD), jnp.float32),
            pltpu.VMEM_SHARED((_NS, 29, 128), jnp.float32),
        ],
        compiler_params=pltpu.CompilerParams(needs_layout_passes=False),
    )


def _preprocess_sc_body(compact_hbm, out_hbm, myv_v, vals_v, spm):
    core = lax.axis_index("c")
    sid = lax.axis_index("s")
    cp = sid & 1
    g = core * 8 + (sid >> 1)
    w_pack = cp * 16 + g

    own = myv_v.at[pl.ds(0, 29)]
    pltpu.sync_copy(compact_hbm.at[w_pack], own)

    lane_iota = lax.iota(jnp.int32, 16)
    even = (lane_iota & 1) == 0
    zeros = jnp.zeros((16,), jnp.float32)

    accs = [zeros, zeros, zeros, zeros]
    for r in range(_N_LM):
        v = myv_v[r >> 3, pl.ds((r & 7) * _L, _L)]
        accs[_seg_of(r)] = accs[_seg_of(r)] + v * v
    rvecs = [_rsqrt_newton(a) for a in accs]

    for r in range(_N_LM):
        sl = (r >> 3, pl.ds((r & 7) * _L, _L))
        y = myv_v[sl] * rvecs[_seg_of(r)]
        myv_v[sl] = jnp.where(y != y, 0.0, y)

    pltpu.sync_copy(own, spm.at[sid])
    plsc.subcore_barrier()
    pltpu.sync_copy(spm.at[sid ^ 1], myv_v.at[pl.ds(_NRB, 29)])

    def frame_body(fl, _):
        for ch in range(_NCHUNK):
            kvec = _L * ch + lane_iota
            r = kvec >> 1
            rows = (r >> 3) + ((kvec & 1) ^ cp) * _NRB
            lanes = (r & 7) * _L + (_FPT * cp + fl)
            vals_v[fl & 7, pl.ds(_L * ch, _L)] = plsc.load_gather(
                myv_v, [rows, lanes])
        return ()

    lax.fori_loop(0, _FPT, frame_body, (), unroll=False)
    pltpu.sync_copy(vals_v, out_hbm.at[pl.ds(16 * g + _FPT * cp, _FPT)])

    @pl.when(cp == 1)
    def _():
        neg1 = jnp.full((16,), -1.0, jnp.float32)
        for r in range(_FPT):
            for ch in range(_NCHUNK):
                vals_v[r, pl.ds(_L * ch, _L)] = neg1
        pltpu.sync_copy(vals_v,
                        out_hbm.at[pl.ds(_N_FRAMES + g * _FPT, _FPT)])


def kernel(data):
    n_frames = data.shape[0]
    data_t = jnp.transpose(data, (1, 2, 0))
    compact = _make_compact_tc()(data_t)
    padded = _make_preprocess_sc()(compact)
    out = padded[:, :_KPF]
    idxs = jnp.concatenate(
        [jnp.arange(n_frames, dtype=jnp.int32),
         jnp.full((_INPUT_SIZE - n_frames,), -1, dtype=jnp.int32)]
    )
    return (out, idxs)
